# Initial kernel scaffold; baseline (speedup 1.0000x reference)
import jax
import jax.numpy as jnp
from jax.experimental import pallas as pl


def kernel(y_pred, y_true, coordinates):
    raise NotImplementedError("write your pallas kernel here")


# speedup vs baseline: 71.0807x; 71.0807x over previous
import numpy as np
import jax
import jax.numpy as jnp
from jax import lax
from jax.experimental import pallas as pl
from jax.experimental.pallas import tpu as pltpu

_N = 1000000
_M = 100
_K = 11
_PAD = 128

_PERM100 = np.asarray(
    jax.random.permutation(jax.random.key(42), _N)[:_M]
).astype(np.int32)

_ROWS_PER_BLOCK = 200
_GRID = 1000 // _ROWS_PER_BLOCK
_INF = float(np.inf)


def _loss_kernel(yp_ref, yt_ref, subr_row_ref, subr_col_ref,
                 cx_row_ref, cx_col_ref, cy_row_ref, cy_col_ref,
                 out_ref, acc_ref):
    step = pl.program_id(0)

    @pl.when(step == 0)
    def _():
        acc_ref[0, 0] = jnp.float32(0.0)

    diff = yp_ref[...] - yt_ref[...]
    acc_ref[0, 0] += jnp.sum(diff * diff)

    @pl.when(step == _GRID - 1)
    def _():
        mse = acc_ref[0, 0] / jnp.float32(_N)

        cx_r = cx_row_ref[...]
        cx_c = cx_col_ref[...]
        cy_r = cy_row_ref[...]
        cy_c = cy_col_ref[...]
        dx = cx_c - cx_r
        dy = cy_c - cy_r
        d = jnp.sqrt(dx * dx + dy * dy)

        iota_j = lax.broadcasted_iota(jnp.int32, (_PAD, _PAD), 1)
        d = jnp.where(iota_j < _M, d, _INF)

        r_row = subr_row_ref[...]
        r_col = subr_col_ref[...]
        r_mat = jnp.broadcast_to(r_row, (_PAD, _PAD))

        picked_d = []
        picked_r = []
        for _t in range(_K):
            m = jnp.min(d, axis=1, keepdims=True)
            cand = jnp.where(d == m, iota_j, jnp.int32(1 << 30))
            jmin = jnp.min(cand, axis=1, keepdims=True)
            onehot = iota_j == jmin
            rsel = jnp.sum(jnp.where(onehot, r_mat, 0.0), axis=1,
                           keepdims=True)
            picked_d.append(m)
            picked_r.append(rsel)
            d = jnp.where(onehot, _INF, d)

        dmax = picked_d[_K - 1]
        pen = jnp.zeros((_PAD, 1), jnp.float32)
        for t in range(1, _K):
            norm = picked_d[t] / (dmax + jnp.float32(1e-8))
            rdiff = jnp.abs(picked_r[t] - r_col)
            pen = pen + jnp.abs(rdiff - norm)
        pen = pen / jnp.float32(_K - 1)

        iota_i = lax.broadcasted_iota(jnp.int32, (_PAD, 1), 0)
        spatial = jnp.sum(jnp.where(iota_i < _M, pen, 0.0)) / jnp.float32(_M)

        out_ref[0, 0] = jnp.float32(0.8) * mse + jnp.float32(0.2) * spatial


def kernel(y_pred, y_true, coordinates):
    idx = jnp.asarray(_PERM100)
    sub_r = (y_pred[idx] - y_true[idx]).astype(jnp.float32)
    sub_c = coordinates[idx]

    def pad128(v):
        return jnp.zeros((_PAD,), jnp.float32).at[:_M].set(v)

    subr = pad128(sub_r)
    cx = pad128(sub_c[:, 0])
    cy = pad128(sub_c[:, 1])

    y2 = y_pred.reshape(1000, 1000)
    t2 = y_true.reshape(1000, 1000)

    full = lambda shape: pl.BlockSpec(shape, lambda i: (0, 0))
    out = pl.pallas_call(
        _loss_kernel,
        grid=(_GRID,),
        in_specs=[
            pl.BlockSpec((_ROWS_PER_BLOCK, 1000), lambda i: (i, 0)),
            pl.BlockSpec((_ROWS_PER_BLOCK, 1000), lambda i: (i, 0)),
            full((1, _PAD)), full((_PAD, 1)),
            full((1, _PAD)), full((_PAD, 1)),
            full((1, _PAD)), full((_PAD, 1)),
        ],
        out_specs=pl.BlockSpec(memory_space=pltpu.SMEM),
        out_shape=jax.ShapeDtypeStruct((1, 1), jnp.float32),
        scratch_shapes=[pltpu.SMEM((1, 1), jnp.float32)],
    )(
        y2, t2,
        subr.reshape(1, _PAD), subr.reshape(_PAD, 1),
        cx.reshape(1, _PAD), cx.reshape(_PAD, 1),
        cy.reshape(1, _PAD), cy.reshape(_PAD, 1),
    )
    return out[0, 0]
